# baseline (speedup 1.0000x reference)
import functools

import jax
import jax.numpy as jnp
from jax import lax
from jax.experimental import pallas as pl
from jax.experimental.pallas import tpu as pltpu
from jax.experimental.pallas import tpu_sc as plsc

N_NODES = 10000
N_EDGES = 320000
D = 128
G = 64
CHUNK = 80
NC = 2
NS = 16
NW = NC * NS
NODE_CHUNKS = N_NODES // CHUNK
GPW = N_EDGES // NW

SPLIT = 102400
EPW = SPLIT // NW
CPW = EPW // CHUNK
TAIL = N_EDGES - SPLIT
BE2 = 4000
BET = 3200


def _gid_body(dst1_hbm, dst2_hbm, batch_hbm, gid1_out, gid2_out,
              batch_v, idx_all, gid_all):
    cid = lax.axis_index("c")
    sid = lax.axis_index("s")
    wid = sid * NC + cid
    wbase = pl.multiple_of(wid * GPW, 8)
    pltpu.sync_copy(batch_hbm, batch_v)

    for dst_hbm, gid_out in ((dst1_hbm, gid1_out), (dst2_hbm, gid2_out)):
        pltpu.sync_copy(dst_hbm.at[pl.ds(wbase, GPW)], idx_all)

        def _g(j, _):
            iv = idx_all[pl.ds(j * 16, 16)]
            gid_all[pl.ds(j * 16, 16)] = plsc.load_gather(batch_v, [iv])
            return 0
        lax.fori_loop(0, GPW // 16, _g, 0)
        pltpu.sync_copy(gid_all, gid_out.at[pl.ds(wbase, GPW)])


_sc_gid = functools.partial(
    pl.kernel,
    out_type=[
        jax.ShapeDtypeStruct((N_EDGES,), jnp.int32),
        jax.ShapeDtypeStruct((N_EDGES,), jnp.int32),
    ],
    mesh=plsc.VectorSubcoreMesh(core_axis_name="c", subcore_axis_name="s"),
    compiler_params=pltpu.CompilerParams(needs_layout_passes=False),
    scratch_types=[
        pltpu.VMEM((N_NODES,), jnp.int32),
        pltpu.VMEM((GPW,), jnp.int32),
        pltpu.VMEM((GPW,), jnp.int32),
    ],
)(_gid_body)


def _sc_body(attr0, attr1, gid1_hbm, batch_hbm,
             partial_out, counts_out,
             gid_flat, gid_e, gid_n, rows_v, rows_b, ones_v,
             zero_v, acc_sh, cnt_sh, sem0, sem1):
    cid = lax.axis_index("c")
    sid = lax.axis_index("s")
    wid = sid * NC + cid

    zf = jnp.zeros((16,), jnp.float32)
    of = jnp.ones((16,), jnp.float32)

    def _zrow(r, _):
        for j in range(D // 16):
            zero_v[r, pl.ds(j * 16, 16)] = zf
        return 0
    lax.fori_loop(0, G, _zrow, 0)

    def _orow(r, _):
        for j in range(D // 16):
            ones_v[r, pl.ds(j * 16, 16)] = of
        return 0
    lax.fori_loop(0, CHUNK, _orow, 0)

    @pl.when(sid == 0)
    def _():
        pltpu.sync_copy(zero_v, acc_sh)
        pltpu.sync_copy(zero_v, cnt_sh)

    pltpu.sync_copy(gid1_hbm.at[pl.ds(pl.multiple_of(wid * EPW, 8), EPW)],
                    gid_flat)

    def _pack(j, _):
        for u in range(CHUNK // 16):
            gid_e[j, pl.ds(u * 16, 16)] = gid_flat[pl.ds(j * CHUNK + u * 16, 16)]
        return 0
    lax.fori_loop(0, CPW, _pack, 0)
    plsc.subcore_barrier()

    wbase = pl.multiple_of(wid * EPW, 8)

    def _fill(buf, sem, ci):
        base = pl.multiple_of(wbase + ci * CHUNK, 8)
        pltpu.async_copy(attr1.at[pl.ds(base, CHUNK)], buf, sem)

    def _wait(buf, sem):
        pltpu.make_async_copy(attr1.at[pl.ds(0, CHUNK)], buf, sem).wait()

    _fill(rows_v, sem0, 0)

    def body(k, _):
        i0 = k * 2
        _wait(rows_v, sem0)
        _fill(rows_b, sem1, i0 + 1)
        pltpu.sync_copy(rows_v, acc_sh.at[gid_e.at[i0]], add=True)
        _wait(rows_b, sem1)
        @pl.when(i0 + 2 < CPW)
        def _():
            _fill(rows_v, sem0, i0 + 2)
        pltpu.sync_copy(rows_b, acc_sh.at[gid_e.at[i0 + 1]], add=True)
        return 0
    lax.fori_loop(0, CPW // 2, body, 0)

    def _node_body(i, _):
        ci = i * NW + wid
        @pl.when(ci < NODE_CHUNKS)
        def _():
            base = pl.multiple_of(ci * CHUNK, 8)
            pltpu.sync_copy(batch_hbm.at[pl.ds(base, CHUNK)], gid_n)
            pltpu.sync_copy(attr0.at[pl.ds(base, CHUNK)], rows_v)
            pltpu.sync_copy(rows_v, acc_sh.at[gid_n], add=True)
            pltpu.sync_copy(ones_v, cnt_sh.at[gid_n], add=True)
        return 0
    lax.fori_loop(0, (NODE_CHUNKS + NW - 1) // NW, _node_body, 0)

    plsc.subcore_barrier()

    @pl.when(sid == 0)
    def _():
        pltpu.sync_copy(acc_sh, partial_out.at[cid])
        pltpu.sync_copy(cnt_sh, counts_out.at[cid])


_sc_pool = functools.partial(
    pl.kernel,
    out_type=[
        jax.ShapeDtypeStruct((NC, G, D), jnp.float32),
        jax.ShapeDtypeStruct((NC, G, D), jnp.float32),
    ],
    mesh=plsc.VectorSubcoreMesh(core_axis_name="c", subcore_axis_name="s"),
    compiler_params=pltpu.CompilerParams(needs_layout_passes=False),
    scratch_types=[
        pltpu.VMEM((EPW,), jnp.int32),
        pltpu.VMEM((CPW, CHUNK), jnp.int32),
        pltpu.VMEM((CHUNK,), jnp.int32),
        pltpu.VMEM((CHUNK, D), jnp.float32),
        pltpu.VMEM((CHUNK, D), jnp.float32),
        pltpu.VMEM((CHUNK, D), jnp.float32),
        pltpu.VMEM((G, D), jnp.float32),
        pltpu.VMEM_SHARED((G, D), jnp.float32),
        pltpu.VMEM_SHARED((G, D), jnp.float32),
        pltpu.SemaphoreType.DMA,
        pltpu.SemaphoreType.DMA,
    ],
)(_sc_body)


def _tc_onehot_body(gid_ref, x_ref, o_ref):
    i = pl.program_id(0)
    gid = gid_ref[0]
    be = gid_ref.shape[2]
    iota = lax.broadcasted_iota(jnp.int32, (G, be), 0)
    onehot = (gid == iota).astype(jnp.float32)
    p = jax.lax.dot_general(onehot, x_ref[...], (((1,), (0,)), ((), ())),
                            preferred_element_type=jnp.float32,
                            precision=jax.lax.Precision.HIGHEST)

    @pl.when(i == 0)
    def _():
        o_ref[...] = jnp.zeros_like(o_ref)
    o_ref[...] += p


def _tc_segment_sum(gid, attr, be, start_block=0):
    n = attr.shape[0]
    nb = n // be - start_block
    gid3 = gid.reshape(n // be, 1, be)
    return pl.pallas_call(
        _tc_onehot_body,
        grid=(nb,),
        in_specs=[
            pl.BlockSpec((1, 1, be), lambda i: (i + start_block, 0, 0)),
            pl.BlockSpec((be, D), lambda i: (i + start_block, 0)),
        ],
        out_specs=pl.BlockSpec((G, D), lambda i: (0, 0)),
        out_shape=jax.ShapeDtypeStruct((G, D), jnp.float32),
    )(gid3, attr)


def _combine_body(p_ref, q_ref, r_ref, c_ref, o_ref):
    s = p_ref[0] + p_ref[1] + q_ref[...] + r_ref[...]
    cnt = c_ref[0, :, 0:1] + c_ref[1, :, 0:1]
    o_ref[...] = s / jnp.maximum(cnt, 1.0)


def kernel(edge_attr0, edge_attr1, edge_attr2, edge_index, edge_index2,
           num_nodes, batch):
    dst1 = edge_index[1].astype(jnp.int32)
    dst2 = edge_index2[1].astype(jnp.int32)
    batch32 = batch.astype(jnp.int32)
    gid1, gid2 = _sc_gid(dst1, dst2, batch32)
    tc2 = _tc_segment_sum(gid2, edge_attr2, BE2)
    tc1 = _tc_segment_sum(gid1, edge_attr1, BET, start_block=SPLIT // BET)
    partial, counts = _sc_pool(edge_attr0, edge_attr1, gid1, batch32)
    out = pl.pallas_call(
        _combine_body,
        out_shape=jax.ShapeDtypeStruct((G, D), jnp.float32),
    )(partial, tc2, tc1, counts)
    return out


# speedup vs baseline: 1.1542x; 1.1542x over previous
import functools

import jax
import jax.numpy as jnp
from jax import lax
from jax.experimental import pallas as pl
from jax.experimental.pallas import tpu as pltpu
from jax.experimental.pallas import tpu_sc as plsc

N_NODES = 10000
N_EDGES = 320000
D = 128
G = 64
CHUNK = 80
NC = 2
NS = 16
NW = NC * NS
NODE_CHUNKS = N_NODES // CHUNK
GPW = N_EDGES // NW

SPLIT = 320000
EPW = SPLIT // NW
CPW = EPW // CHUNK
TAIL = N_EDGES - SPLIT
BE2 = 4000
BET = 3200


def _gid_body(dst1_hbm, dst2_hbm, batch_hbm, gid1_out, gid2_out,
              batch_v, idx_all, gid_all):
    cid = lax.axis_index("c")
    sid = lax.axis_index("s")
    wid = sid * NC + cid
    wbase = pl.multiple_of(wid * GPW, 8)
    pltpu.sync_copy(batch_hbm, batch_v)

    for dst_hbm, gid_out in ((dst1_hbm, gid1_out), (dst2_hbm, gid2_out)):
        pltpu.sync_copy(dst_hbm.at[pl.ds(wbase, GPW)], idx_all)

        def _g(j, _):
            iv = idx_all[pl.ds(j * 16, 16)]
            gid_all[pl.ds(j * 16, 16)] = plsc.load_gather(batch_v, [iv])
            return 0
        lax.fori_loop(0, GPW // 16, _g, 0)
        pltpu.sync_copy(gid_all, gid_out.at[pl.ds(wbase, GPW)])


_sc_gid = functools.partial(
    pl.kernel,
    out_type=[
        jax.ShapeDtypeStruct((N_EDGES,), jnp.int32),
        jax.ShapeDtypeStruct((N_EDGES,), jnp.int32),
    ],
    mesh=plsc.VectorSubcoreMesh(core_axis_name="c", subcore_axis_name="s"),
    compiler_params=pltpu.CompilerParams(needs_layout_passes=False),
    scratch_types=[
        pltpu.VMEM((N_NODES,), jnp.int32),
        pltpu.VMEM((GPW,), jnp.int32),
        pltpu.VMEM((GPW,), jnp.int32),
    ],
)(_gid_body)


def _sc_body(attr0, attr1, gid1_hbm, batch_hbm,
             partial_out, counts_out,
             gid_flat, gid_e, gid_n, rows_v, rows_b, ones_v,
             zero_v, acc_sh, cnt_sh, sem0, sem1):
    cid = lax.axis_index("c")
    sid = lax.axis_index("s")
    wid = sid * NC + cid

    zf = jnp.zeros((16,), jnp.float32)
    of = jnp.ones((16,), jnp.float32)

    def _zrow(r, _):
        for j in range(D // 16):
            zero_v[r, pl.ds(j * 16, 16)] = zf
        return 0
    lax.fori_loop(0, G, _zrow, 0)

    def _orow(r, _):
        for j in range(D // 16):
            ones_v[r, pl.ds(j * 16, 16)] = of
        return 0
    lax.fori_loop(0, CHUNK, _orow, 0)

    @pl.when(sid == 0)
    def _():
        pltpu.sync_copy(zero_v, acc_sh)
        pltpu.sync_copy(zero_v, cnt_sh)

    pltpu.sync_copy(gid1_hbm.at[pl.ds(pl.multiple_of(wid * EPW, 8), EPW)],
                    gid_flat)

    def _pack(j, _):
        for u in range(CHUNK // 16):
            gid_e[j, pl.ds(u * 16, 16)] = gid_flat[pl.ds(j * CHUNK + u * 16, 16)]
        return 0
    lax.fori_loop(0, CPW, _pack, 0)
    plsc.subcore_barrier()

    wbase = pl.multiple_of(wid * EPW, 8)

    def _fill(buf, sem, ci):
        base = pl.multiple_of(wbase + ci * CHUNK, 8)
        pltpu.async_copy(attr1.at[pl.ds(base, CHUNK)], buf, sem)

    def _wait(buf, sem):
        pltpu.make_async_copy(attr1.at[pl.ds(0, CHUNK)], buf, sem).wait()

    _fill(rows_v, sem0, 0)

    def body(k, _):
        i0 = k * 2
        _wait(rows_v, sem0)
        _fill(rows_b, sem1, i0 + 1)
        pltpu.sync_copy(rows_v, acc_sh.at[gid_e.at[i0]], add=True)
        _wait(rows_b, sem1)
        @pl.when(i0 + 2 < CPW)
        def _():
            _fill(rows_v, sem0, i0 + 2)
        pltpu.sync_copy(rows_b, acc_sh.at[gid_e.at[i0 + 1]], add=True)
        return 0
    lax.fori_loop(0, CPW // 2, body, 0)
    if CPW % 2:
        _wait(rows_v, sem0)
        pltpu.sync_copy(rows_v, acc_sh.at[gid_e.at[CPW - 1]], add=True)

    def _node_body(i, _):
        ci = i * NW + wid
        @pl.when(ci < NODE_CHUNKS)
        def _():
            base = pl.multiple_of(ci * CHUNK, 8)
            pltpu.sync_copy(batch_hbm.at[pl.ds(base, CHUNK)], gid_n)
            pltpu.sync_copy(attr0.at[pl.ds(base, CHUNK)], rows_v)
            pltpu.sync_copy(rows_v, acc_sh.at[gid_n], add=True)
            pltpu.sync_copy(ones_v, cnt_sh.at[gid_n], add=True)
        return 0
    lax.fori_loop(0, (NODE_CHUNKS + NW - 1) // NW, _node_body, 0)

    plsc.subcore_barrier()

    @pl.when(sid == 0)
    def _():
        pltpu.sync_copy(acc_sh, partial_out.at[cid])
        pltpu.sync_copy(cnt_sh, counts_out.at[cid])


_sc_pool = functools.partial(
    pl.kernel,
    out_type=[
        jax.ShapeDtypeStruct((NC, G, D), jnp.float32),
        jax.ShapeDtypeStruct((NC, G, D), jnp.float32),
    ],
    mesh=plsc.VectorSubcoreMesh(core_axis_name="c", subcore_axis_name="s"),
    compiler_params=pltpu.CompilerParams(needs_layout_passes=False),
    scratch_types=[
        pltpu.VMEM((EPW,), jnp.int32),
        pltpu.VMEM((CPW, CHUNK), jnp.int32),
        pltpu.VMEM((CHUNK,), jnp.int32),
        pltpu.VMEM((CHUNK, D), jnp.float32),
        pltpu.VMEM((CHUNK, D), jnp.float32),
        pltpu.VMEM((CHUNK, D), jnp.float32),
        pltpu.VMEM((G, D), jnp.float32),
        pltpu.VMEM_SHARED((G, D), jnp.float32),
        pltpu.VMEM_SHARED((G, D), jnp.float32),
        pltpu.SemaphoreType.DMA,
        pltpu.SemaphoreType.DMA,
    ],
)(_sc_body)


def _tc_onehot_body(gid_ref, x_ref, o_ref):
    i = pl.program_id(0)
    gid = gid_ref[0]
    be = gid_ref.shape[2]
    iota = lax.broadcasted_iota(jnp.int32, (G, be), 0)
    onehot = (gid == iota).astype(jnp.float32)
    p = jax.lax.dot_general(onehot, x_ref[...], (((1,), (0,)), ((), ())),
                            preferred_element_type=jnp.float32,
                            precision=jax.lax.Precision.HIGHEST)

    @pl.when(i == 0)
    def _():
        o_ref[...] = jnp.zeros_like(o_ref)
    o_ref[...] += p


def _tc_segment_sum(gid, attr, be, start_block=0):
    n = attr.shape[0]
    nb = n // be - start_block
    gid3 = gid.reshape(n // be, 1, be)
    return pl.pallas_call(
        _tc_onehot_body,
        grid=(nb,),
        in_specs=[
            pl.BlockSpec((1, 1, be), lambda i: (i + start_block, 0, 0)),
            pl.BlockSpec((be, D), lambda i: (i + start_block, 0)),
        ],
        out_specs=pl.BlockSpec((G, D), lambda i: (0, 0)),
        out_shape=jax.ShapeDtypeStruct((G, D), jnp.float32),
    )(gid3, attr)


def _combine_body(p_ref, *rest):
    *q_refs, c_ref, o_ref = rest
    s = p_ref[0] + p_ref[1]
    for q in q_refs:
        s = s + q[...]
    cnt = c_ref[0, :, 0:1] + c_ref[1, :, 0:1]
    o_ref[...] = s / jnp.maximum(cnt, 1.0)


def kernel(edge_attr0, edge_attr1, edge_attr2, edge_index, edge_index2,
           num_nodes, batch):
    dst1 = edge_index[1].astype(jnp.int32)
    dst2 = edge_index2[1].astype(jnp.int32)
    batch32 = batch.astype(jnp.int32)
    gid1, gid2 = _sc_gid(dst1, dst2, batch32)
    tc_parts = [_tc_segment_sum(gid2, edge_attr2, BE2)]
    if TAIL:
        tc_parts.append(
            _tc_segment_sum(gid1, edge_attr1, BET, start_block=SPLIT // BET))
    partial, counts = _sc_pool(edge_attr0, edge_attr1, gid1, batch32)
    out = pl.pallas_call(
        _combine_body,
        out_shape=jax.ShapeDtypeStruct((G, D), jnp.float32),
    )(partial, *tc_parts, counts)
    return out
